# initial kernel scaffold (unmeasured)
import jax
import jax.numpy as jnp
from jax import lax
from jax.experimental import pallas as pl
from jax.experimental.pallas import tpu as pltpu

M_TOTAL = 8192
OUT_M = 4096
D = 4096
CHUNK = 512
N_CHUNKS = OUT_M // CHUNK


def kernel(partial, gamma):
    p2d = partial.reshape(M_TOTAL, D)
    g2d = gamma.reshape(1, D)

    def body(p_ref, g_ref, out_ref, recv_ref, a_ref, b_ref, o_ref,
             local_sems, send_sem, recv_sem):
        my_x = lax.axis_index("x")
        my_y = lax.axis_index("y")
        peer_y = 1 - my_y

        barrier = pltpu.get_barrier_semaphore()
        pl.semaphore_signal(barrier, inc=1, device_id=(my_x, peer_y),
                            device_id_type=pl.DeviceIdType.MESH)
        pl.semaphore_wait(barrier, 1)

        rdma = pltpu.make_async_remote_copy(
            src_ref=p_ref.at[pl.ds(peer_y * OUT_M, OUT_M)],
            dst_ref=recv_ref,
            send_sem=send_sem,
            recv_sem=recv_sem,
            device_id=(my_x, peer_y),
            device_id_type=pl.DeviceIdType.MESH,
        )
        rdma.start()
        rdma.wait()

        def chunk_body(c, carry):
            row = c * CHUNK
            cp_a = pltpu.make_async_copy(
                p_ref.at[pl.ds(my_y * OUT_M + row, CHUNK)], a_ref,
                local_sems.at[0])
            cp_b = pltpu.make_async_copy(
                recv_ref.at[pl.ds(row, CHUNK)], b_ref, local_sems.at[1])
            cp_a.start()
            cp_b.start()
            cp_a.wait()
            cp_b.wait()
            y = a_ref[...] + b_ref[...]
            ms = jnp.mean(y * y, axis=-1, keepdims=True)
            o_ref[...] = y * lax.rsqrt(ms + 1e-6) * g_ref[...]
            cp_o = pltpu.make_async_copy(
                o_ref, out_ref.at[pl.ds(row, CHUNK)], local_sems.at[2])
            cp_o.start()
            cp_o.wait()
            return carry

        lax.fori_loop(0, N_CHUNKS, chunk_body, 0)

    return pl.pallas_call(
        body,
        out_shape=jax.ShapeDtypeStruct((OUT_M, D), jnp.float32),
        in_specs=[
            pl.BlockSpec(memory_space=pl.ANY),
            pl.BlockSpec(memory_space=pltpu.MemorySpace.VMEM),
        ],
        out_specs=pl.BlockSpec(memory_space=pl.ANY),
        scratch_shapes=[
            pltpu.MemorySpace.HBM((OUT_M, D), jnp.float32),
            pltpu.VMEM((CHUNK, D), jnp.float32),
            pltpu.VMEM((CHUNK, D), jnp.float32),
            pltpu.VMEM((CHUNK, D), jnp.float32),
            pltpu.SemaphoreType.DMA((3,)),
            pltpu.SemaphoreType.DMA,
            pltpu.SemaphoreType.DMA,
        ],
        compiler_params=pltpu.CompilerParams(collective_id=0),
    )(p2d, g2d)


# baseline (device time: 859184 ns/iter reference)
import jax
import jax.numpy as jnp
from jax import lax
from jax.experimental import pallas as pl
from jax.experimental.pallas import tpu as pltpu

M_TOTAL = 8192
OUT_M = 4096
D = 4096
CHUNK = 512
N_CHUNKS = OUT_M // CHUNK


def kernel(partial, gamma):
    p2d = partial.reshape(M_TOTAL, D)
    g2d = gamma.reshape(1, D)

    def body(p_ref, g_ref, out_ref, recv_ref,
             a_ref, b_ref, o_ref, local_sems, send_sem, recv_sem):
        my_x = lax.axis_index("x")
        my_y = lax.axis_index("y")
        peer_y = 1 - my_y

        barrier = pltpu.get_barrier_semaphore()
        pl.semaphore_signal(barrier, inc=1, device_id=(my_x, peer_y),
                            device_id_type=pl.DeviceIdType.MESH)
        pl.semaphore_wait(barrier, 1)

        rdma = pltpu.make_async_remote_copy(
            src_ref=p_ref.at[pl.ds(peer_y * OUT_M, OUT_M)],
            dst_ref=recv_ref,
            send_sem=send_sem,
            recv_sem=recv_sem,
            device_id=(my_x, peer_y),
            device_id_type=pl.DeviceIdType.MESH,
        )
        rdma.start()
        rdma.wait()

        def chunk_body(c, carry):
            row = c * CHUNK
            cp_a = pltpu.make_async_copy(
                p_ref.at[pl.ds(my_y * OUT_M + row, CHUNK)], a_ref,
                local_sems.at[0])
            cp_b = pltpu.make_async_copy(
                recv_ref.at[pl.ds(row, CHUNK)], b_ref, local_sems.at[1])
            cp_a.start()
            cp_b.start()
            cp_a.wait()
            cp_b.wait()
            y = a_ref[...] + b_ref[...]
            ms = jnp.mean(y * y, axis=-1, keepdims=True)
            o_ref[...] = y * lax.rsqrt(ms + 1e-6) * g_ref[...]
            cp_o = pltpu.make_async_copy(
                o_ref, out_ref.at[pl.ds(row, CHUNK)], local_sems.at[2])
            cp_o.start()
            cp_o.wait()
            return carry

        lax.fori_loop(0, N_CHUNKS, chunk_body, 0)

    out, _recv = pl.pallas_call(
        body,
        out_shape=[
            jax.ShapeDtypeStruct((OUT_M, D), jnp.float32),
            jax.ShapeDtypeStruct((OUT_M, D), jnp.float32),
        ],
        in_specs=[
            pl.BlockSpec(memory_space=pl.ANY),
            pl.BlockSpec(memory_space=pltpu.MemorySpace.VMEM),
        ],
        out_specs=[
            pl.BlockSpec(memory_space=pl.ANY),
            pl.BlockSpec(memory_space=pl.ANY),
        ],
        scratch_shapes=[
            pltpu.VMEM((CHUNK, D), jnp.float32),
            pltpu.VMEM((CHUNK, D), jnp.float32),
            pltpu.VMEM((CHUNK, D), jnp.float32),
            pltpu.SemaphoreType.DMA((3,)),
            pltpu.SemaphoreType.DMA,
            pltpu.SemaphoreType.DMA,
        ],
        compiler_params=pltpu.CompilerParams(
            collective_id=0,
            vmem_limit_bytes=64 * 1024 * 1024,
        ),
    )(p2d, g2d)
    return out


# device time: 438638 ns/iter; 1.9588x vs baseline; 1.9588x over previous
import jax
import jax.numpy as jnp
from jax import lax
from jax.experimental import pallas as pl
from jax.experimental.pallas import tpu as pltpu

M_TOTAL = 8192
OUT_M = 4096
HALF = 2048
D = 4096
CHUNK = 128
N_CHUNKS = HALF // CHUNK


def kernel(partial, gamma):
    p2d = partial.reshape(M_TOTAL, D)
    g2d = gamma.reshape(1, D)

    def body(p_ref, g_ref, out_ref, recv_ref,
             a_ref, b_ref, o_ref, local_sems,
             y_send_sems, y_recv_sems, x_send_sems, x_recv_sems):
        my_x = lax.axis_index("x")
        my_y = lax.axis_index("y")
        y_peer = (my_x, 1 - my_y)
        x_peer = (1 - my_x, my_y)

        mine0 = my_y * OUT_M + my_x * HALF
        send0 = (1 - my_y) * OUT_M + my_x * HALF
        out0 = my_x * HALF
        theirs0 = (1 - my_x) * HALF

        barrier = pltpu.get_barrier_semaphore()
        for peer in (y_peer, x_peer):
            pl.semaphore_signal(barrier, inc=1, device_id=peer,
                                device_id_type=pl.DeviceIdType.MESH)
        pl.semaphore_wait(barrier, 2)

        for c in range(N_CHUNKS):
            pltpu.make_async_remote_copy(
                src_ref=p_ref.at[pl.ds(send0 + c * CHUNK, CHUNK)],
                dst_ref=recv_ref.at[pl.ds(c * CHUNK, CHUNK)],
                send_sem=y_send_sems.at[c],
                recv_sem=y_recv_sems.at[c],
                device_id=y_peer,
                device_id_type=pl.DeviceIdType.MESH,
            ).start()

        for c in range(N_CHUNKS):
            cp_a = pltpu.make_async_copy(
                p_ref.at[pl.ds(mine0 + c * CHUNK, CHUNK)], a_ref,
                local_sems.at[0])
            cp_a.start()
            pltpu.make_async_remote_copy(
                src_ref=p_ref.at[pl.ds(send0 + c * CHUNK, CHUNK)],
                dst_ref=recv_ref.at[pl.ds(c * CHUNK, CHUNK)],
                send_sem=y_send_sems.at[c],
                recv_sem=y_recv_sems.at[c],
                device_id=y_peer,
                device_id_type=pl.DeviceIdType.MESH,
            ).wait_recv()
            cp_b = pltpu.make_async_copy(
                recv_ref.at[pl.ds(c * CHUNK, CHUNK)], b_ref,
                local_sems.at[1])
            cp_b.start()
            cp_a.wait()
            cp_b.wait()
            y = a_ref[...] + b_ref[...]
            ms = jnp.mean(y * y, axis=-1, keepdims=True)
            o_ref[...] = y * lax.rsqrt(ms + 1e-6) * g_ref[...]
            cp_o = pltpu.make_async_copy(
                o_ref, out_ref.at[pl.ds(out0 + c * CHUNK, CHUNK)],
                local_sems.at[2])
            cp_o.start()
            cp_o.wait()
            pltpu.make_async_remote_copy(
                src_ref=out_ref.at[pl.ds(out0 + c * CHUNK, CHUNK)],
                dst_ref=out_ref.at[pl.ds(out0 + c * CHUNK, CHUNK)],
                send_sem=x_send_sems.at[c],
                recv_sem=x_recv_sems.at[c],
                device_id=x_peer,
                device_id_type=pl.DeviceIdType.MESH,
            ).start()

        for c in range(N_CHUNKS):
            pltpu.make_async_remote_copy(
                src_ref=out_ref.at[pl.ds(out0 + c * CHUNK, CHUNK)],
                dst_ref=out_ref.at[pl.ds(theirs0 + c * CHUNK, CHUNK)],
                send_sem=x_send_sems.at[c],
                recv_sem=x_recv_sems.at[c],
                device_id=x_peer,
                device_id_type=pl.DeviceIdType.MESH,
            ).wait_recv()
        for c in range(N_CHUNKS):
            pltpu.make_async_remote_copy(
                src_ref=p_ref.at[pl.ds(send0 + c * CHUNK, CHUNK)],
                dst_ref=recv_ref.at[pl.ds(c * CHUNK, CHUNK)],
                send_sem=y_send_sems.at[c],
                recv_sem=y_recv_sems.at[c],
                device_id=y_peer,
                device_id_type=pl.DeviceIdType.MESH,
            ).wait_send()
            pltpu.make_async_remote_copy(
                src_ref=out_ref.at[pl.ds(out0 + c * CHUNK, CHUNK)],
                dst_ref=out_ref.at[pl.ds(out0 + c * CHUNK, CHUNK)],
                send_sem=x_send_sems.at[c],
                recv_sem=x_recv_sems.at[c],
                device_id=x_peer,
                device_id_type=pl.DeviceIdType.MESH,
            ).wait_send()

    out, _recv = pl.pallas_call(
        body,
        out_shape=[
            jax.ShapeDtypeStruct((OUT_M, D), jnp.float32),
            jax.ShapeDtypeStruct((HALF, D), jnp.float32),
        ],
        in_specs=[
            pl.BlockSpec(memory_space=pl.ANY),
            pl.BlockSpec(memory_space=pltpu.MemorySpace.VMEM),
        ],
        out_specs=[
            pl.BlockSpec(memory_space=pl.ANY),
            pl.BlockSpec(memory_space=pl.ANY),
        ],
        scratch_shapes=[
            pltpu.VMEM((CHUNK, D), jnp.float32),
            pltpu.VMEM((CHUNK, D), jnp.float32),
            pltpu.VMEM((CHUNK, D), jnp.float32),
            pltpu.SemaphoreType.DMA((3,)),
            pltpu.SemaphoreType.DMA((N_CHUNKS,)),
            pltpu.SemaphoreType.DMA((N_CHUNKS,)),
            pltpu.SemaphoreType.DMA((N_CHUNKS,)),
            pltpu.SemaphoreType.DMA((N_CHUNKS,)),
        ],
        compiler_params=pltpu.CompilerParams(
            collective_id=0,
            vmem_limit_bytes=64 * 1024 * 1024,
        ),
    )(p2d, g2d)
    return out
